# Initial kernel scaffold; baseline (speedup 1.0000x reference)
import functools

import jax
import jax.numpy as jnp
from jax import lax
from jax.experimental import pallas as pl
from jax.experimental.pallas import tpu as pltpu
from jax.experimental.pallas import tpu_sc as plsc

N = 10000
E = 320000
D = 128

NC = 2
NS = 16
NW = NC * NS
EPW = E // NW
CH = 80
NCHUNK = EPW // CH
RPT = N // NS
DEGW = 16


def _sc_aggregate(x, src, dst, zeros_nd, zeros_ndeg, ones_ch):
  mesh = plsc.VectorSubcoreMesh(core_axis_name="c", subcore_axis_name="s")

  @functools.partial(
      pl.kernel,
      out_type=(
          jax.ShapeDtypeStruct((NC * N, D), jnp.float32),
          jax.ShapeDtypeStruct((NC * N, DEGW), jnp.float32),
      ),
      mesh=mesh,
      scratch_types=[
          pltpu.VMEM((CH,), jnp.int32),
          pltpu.VMEM((CH,), jnp.int32),
          pltpu.VMEM((CH, D), jnp.float32),
          pltpu.VMEM((CH, DEGW), jnp.float32),
          pltpu.VMEM_SHARED((N, D), jnp.float32),
          pltpu.VMEM_SHARED((N, DEGW), jnp.float32),
          pltpu.SemaphoreType.DMA,
      ],
  )
  def body(x_hbm, src_hbm, dst_hbm, znd_hbm, zdeg_hbm, ones_hbm,
           agg_hbm, deg_hbm,
           src_v, dst_v, rows_v, ones_v, agg_sh, deg_sh, sem):
    c = lax.axis_index("c")
    s = lax.axis_index("s")
    wid = s * NC + c

    pltpu.sync_copy(ones_hbm, ones_v)
    r0 = s * RPT
    pltpu.sync_copy(znd_hbm.at[pl.ds(r0, RPT)], agg_sh.at[pl.ds(r0, RPT)])
    pltpu.sync_copy(zdeg_hbm.at[pl.ds(r0, RPT)], deg_sh.at[pl.ds(r0, RPT)])
    plsc.subcore_barrier()

    base0 = wid * EPW

    def chunk_body(i, carry):
      base = base0 + i * CH
      pltpu.sync_copy(src_hbm.at[pl.ds(base, CH)], src_v)
      pltpu.sync_copy(dst_hbm.at[pl.ds(base, CH)], dst_v)
      pltpu.async_copy(x_hbm.at[src_v], rows_v, sem).wait()
      pltpu.sync_copy(rows_v, agg_sh.at[dst_v], add=True)
      pltpu.sync_copy(ones_v, deg_sh.at[dst_v], add=True)
      return carry

    lax.fori_loop(0, NCHUNK, chunk_body, 0)
    plsc.subcore_barrier()

    out_r0 = c * N + r0
    pltpu.sync_copy(agg_sh.at[pl.ds(r0, RPT)], agg_hbm.at[pl.ds(out_r0, RPT)])
    pltpu.sync_copy(deg_sh.at[pl.ds(r0, RPT)], deg_hbm.at[pl.ds(out_r0, RPT)])

  return body(x, src, dst, zeros_nd, zeros_ndeg, ones_ch)


BR = 1000


def _tc_combine(x, agg0, agg1, deg0, deg1, W_self, W_neigh, b_self, b_neigh):
  def body(x_ref, a0_ref, a1_ref, d0_ref, d1_ref, ws_ref, wn_ref,
           bs_ref, bn_ref, o_ref):
    deg = d0_ref[...][:, 0:1] + d1_ref[...][:, 0:1]
    deg = jnp.maximum(deg, 1.0)
    agg = (a0_ref[...] + a1_ref[...]) / deg
    h = lax.dot_general(x_ref[...], ws_ref[...], (((1,), (1,)), ((), ())),
                        preferred_element_type=jnp.float32)
    h = h + lax.dot_general(agg, wn_ref[...], (((1,), (1,)), ((), ())),
                            preferred_element_type=jnp.float32)
    h = h + bs_ref[...] + bn_ref[...]
    o_ref[...] = jnp.maximum(h, 0.0)

  nb = N // BR
  row_blk = lambda i: (i, 0)
  fixed = lambda i: (0, 0)
  return pl.pallas_call(
      body,
      grid=(nb,),
      in_specs=[
          pl.BlockSpec((BR, D), row_blk),
          pl.BlockSpec((BR, D), row_blk),
          pl.BlockSpec((BR, D), row_blk),
          pl.BlockSpec((BR, DEGW), row_blk),
          pl.BlockSpec((BR, DEGW), row_blk),
          pl.BlockSpec((D, D), fixed),
          pl.BlockSpec((D, D), fixed),
          pl.BlockSpec((1, D), fixed),
          pl.BlockSpec((1, D), fixed),
      ],
      out_specs=pl.BlockSpec((BR, D), row_blk),
      out_shape=jax.ShapeDtypeStruct((N, D), jnp.float32),
  )(x, agg0, agg1, deg0, deg1, W_self, W_neigh,
    b_self[None, :], b_neigh[None, :])


def kernel(x, edge_index, W_self, b_self, W_neigh, b_neigh):
  src = edge_index[0]
  dst = edge_index[1]
  zeros_nd = jnp.zeros((N, D), jnp.float32)
  zeros_ndeg = jnp.zeros((N, DEGW), jnp.float32)
  ones_ch = jnp.ones((CH, DEGW), jnp.float32)
  agg_parts, deg_parts = _sc_aggregate(x, src, dst, zeros_nd, zeros_ndeg,
                                       ones_ch)
  return _tc_combine(x, agg_parts[:N], agg_parts[N:],
                     deg_parts[:N], deg_parts[N:],
                     W_self, W_neigh, b_self, b_neigh)


# speedup vs baseline: 4.8547x; 4.8547x over previous
import functools

import jax
import jax.numpy as jnp
from jax import lax
from jax.experimental import pallas as pl
from jax.experimental.pallas import tpu as pltpu
from jax.experimental.pallas import tpu_sc as plsc

N = 10000
E = 320000
D = 128

NC = 2
NS = 16
NW = NC * NS
EPW = E // NW
CH = 80
NCHUNK = EPW // CH
RCH = 80
NRCH = N // RCH


def _sc_aggregate(x, src, dst, zeros_nd):
  mesh = plsc.VectorSubcoreMesh(core_axis_name="c", subcore_axis_name="s")

  @functools.partial(
      pl.kernel,
      out_type=jax.ShapeDtypeStruct((NC * N, D), jnp.float32),
      mesh=mesh,
      scratch_types=[
          pltpu.VMEM((CH,), jnp.int32),
          pltpu.VMEM((CH,), jnp.int32),
          pltpu.VMEM((CH, D), jnp.float32),
          pltpu.VMEM((RCH, D), jnp.float32),
          pltpu.VMEM_SHARED((N, D), jnp.float32),
          pltpu.SemaphoreType.DMA,
      ],
  )
  def body(x_hbm, src_hbm, dst_hbm, znd_hbm, agg_hbm,
           src_v, dst_v, rows_v, stage_v, agg_sh, sem):
    c = lax.axis_index("c")
    s = lax.axis_index("s")
    wid = s * NC + c

    pltpu.sync_copy(znd_hbm.at[pl.ds(0, RCH)], stage_v)
    nch = 7 + jnp.where(s < NRCH - 7 * NS, 1, 0)

    def init_chunk(k, carry):
      pltpu.sync_copy(stage_v, agg_sh.at[pl.ds((s + k * NS) * RCH, RCH)])
      return carry

    lax.fori_loop(0, nch, init_chunk, 0)
    plsc.subcore_barrier()

    base0 = wid * EPW

    def chunk_body(i, carry):
      base = base0 + i * CH
      pltpu.sync_copy(src_hbm.at[pl.ds(base, CH)], src_v)
      pltpu.sync_copy(dst_hbm.at[pl.ds(base, CH)], dst_v)
      pltpu.async_copy(x_hbm.at[src_v], rows_v, sem).wait()
      pltpu.sync_copy(rows_v, agg_sh.at[dst_v], add=True)
      return carry

    lax.fori_loop(0, NCHUNK, chunk_body, 0)
    plsc.subcore_barrier()

    def pub_chunk(k, carry):
      r = (s + k * NS) * RCH
      pltpu.sync_copy(agg_sh.at[pl.ds(r, RCH)], stage_v)
      pltpu.sync_copy(stage_v, agg_hbm.at[pl.ds(c * N + r, RCH)])
      return carry

    lax.fori_loop(0, nch, pub_chunk, 0)

  return body(x, src, dst, zeros_nd)


EC = 2000


def _tc_degree(dst_col):
  def body(d_ref, o_ref):
    i = pl.program_id(0)
    d = d_ref[...]
    lanes = lax.broadcasted_iota(jnp.int32, (1, 128), 1)
    hi_oh = (lax.shift_right_logical(d, 7) == lanes).astype(jnp.bfloat16)
    lo_oh = (lax.bitwise_and(d, 127) == lanes).astype(jnp.bfloat16)
    part = lax.dot_general(hi_oh, lo_oh, (((0,), (0,)), ((), ())),
                           preferred_element_type=jnp.float32)

    @pl.when(i == 0)
    def _first():
      o_ref[...] = part

    @pl.when(i > 0)
    def _rest():
      o_ref[...] += part

  return pl.pallas_call(
      body,
      grid=(E // EC,),
      in_specs=[pl.BlockSpec((EC, 1), lambda i: (i, 0))],
      out_specs=pl.BlockSpec((128, 128), lambda i: (0, 0)),
      out_shape=jax.ShapeDtypeStruct((128, 128), jnp.float32),
  )(dst_col)


BR = 1000


def _tc_combine(x, agg0, agg1, deg_col, W_self, W_neigh, b_self, b_neigh):
  def body(x_ref, a0_ref, a1_ref, d_ref, ws_ref, wn_ref,
           bs_ref, bn_ref, o_ref):
    deg = jnp.maximum(d_ref[...], 1.0)
    agg = (a0_ref[...] + a1_ref[...]) / deg
    h = lax.dot_general(x_ref[...], ws_ref[...], (((1,), (1,)), ((), ())),
                        preferred_element_type=jnp.float32)
    h = h + lax.dot_general(agg, wn_ref[...], (((1,), (1,)), ((), ())),
                            preferred_element_type=jnp.float32)
    h = h + bs_ref[...] + bn_ref[...]
    o_ref[...] = jnp.maximum(h, 0.0)

  row_blk = lambda i: (i, 0)
  fixed = lambda i: (0, 0)
  return pl.pallas_call(
      body,
      grid=(N // BR,),
      in_specs=[
          pl.BlockSpec((BR, D), row_blk),
          pl.BlockSpec((BR, D), row_blk),
          pl.BlockSpec((BR, D), row_blk),
          pl.BlockSpec((BR, 1), row_blk),
          pl.BlockSpec((D, D), fixed),
          pl.BlockSpec((D, D), fixed),
          pl.BlockSpec((1, D), fixed),
          pl.BlockSpec((1, D), fixed),
      ],
      out_specs=pl.BlockSpec((BR, D), row_blk),
      out_shape=jax.ShapeDtypeStruct((N, D), jnp.float32),
  )(x, agg0, agg1, deg_col, W_self, W_neigh,
    b_self[None, :], b_neigh[None, :])


def kernel(x, edge_index, W_self, b_self, W_neigh, b_neigh):
  src = edge_index[0]
  dst = edge_index[1]
  zeros_nd = jnp.zeros((N, D), jnp.float32)
  agg_parts = _sc_aggregate(x, src, dst, zeros_nd)
  degfold = _tc_degree(dst[:, None])
  deg_col = degfold.reshape(-1)[:N][:, None]
  return _tc_combine(x, agg_parts[:N], agg_parts[N:], deg_col,
                     W_self, W_neigh, b_self, b_neigh)


# speedup vs baseline: 5.8997x; 1.2153x over previous
import functools

import jax
import jax.numpy as jnp
from jax import lax
from jax.experimental import pallas as pl
from jax.experimental.pallas import tpu as pltpu
from jax.experimental.pallas import tpu_sc as plsc

N = 10000
E = 320000
D = 128

NC = 2
NS = 16
NW = NC * NS
EPW = E // NW
CH = 80
NCHUNK = EPW // CH
RCH = 80
NRCH = N // RCH


def _sc_aggregate(x, src, dst3, zeros_nd):
  mesh = plsc.VectorSubcoreMesh(core_axis_name="c", subcore_axis_name="s")

  @functools.partial(
      pl.kernel,
      out_type=jax.ShapeDtypeStruct((NC * N, D), jnp.float32),
      mesh=mesh,
      scratch_types=[
          pltpu.VMEM((CH,), jnp.int32),
          pltpu.VMEM((CH,), jnp.int32),
          pltpu.VMEM((NCHUNK, CH), jnp.int32),
          pltpu.VMEM((CH, D), jnp.float32),
          pltpu.VMEM((CH, D), jnp.float32),
          pltpu.VMEM_SHARED((N, D), jnp.float32),
          pltpu.SemaphoreType.DMA,
          pltpu.SemaphoreType.DMA,
          pltpu.SemaphoreType.DMA,
          pltpu.SemaphoreType.DMA,
      ],
  )
  def body(x_hbm, src_hbm, dst_hbm, znd_hbm, agg_hbm,
           src_ia, src_ib, dsts_v, rows_a, rows_b, agg_sh,
           sem_a, sem_b, sem_ia, sem_ib):
    c = lax.axis_index("c")
    s = lax.axis_index("s")
    wid = s * NC + c

    pltpu.sync_copy(dst_hbm.at[wid], dsts_v)

    pltpu.sync_copy(znd_hbm.at[pl.ds(0, RCH)], rows_b)
    nch = 7 + jnp.where(s < NRCH - 7 * NS, 1, 0)

    def init_chunk(k, carry):
      pltpu.sync_copy(rows_b, agg_sh.at[pl.ds((s + k * NS) * RCH, RCH)])
      return carry

    lax.fori_loop(0, nch, init_chunk, 0)
    plsc.subcore_barrier()

    base0 = wid * EPW

    def fetch_src(ci, idx_v, sem):
      pltpu.async_copy(src_hbm.at[pl.ds(base0 + ci * CH, CH)], idx_v, sem)

    def wait_src(idx_v, sem):
      pltpu.make_async_copy(src_hbm.at[pl.ds(base0, CH)], idx_v, sem).wait()

    def gather(idx_v, rows, sem):
      pltpu.async_copy(x_hbm.at[idx_v], rows, sem)

    def wait_gather(idx_v, rows, sem):
      pltpu.make_async_copy(x_hbm.at[idx_v], rows, sem).wait()

    def scatter(ci, rows):
      pltpu.sync_copy(rows, agg_sh.at[dsts_v.at[ci]], add=True)

    pltpu.sync_copy(src_hbm.at[pl.ds(base0, CH)], src_ia)
    gather(src_ia, rows_a, sem_a)
    fetch_src(1, src_ib, sem_ib)

    def pair_body(j, carry):
      c0 = 2 * j
      wait_gather(src_ia, rows_a, sem_a)
      wait_src(src_ib, sem_ib)
      gather(src_ib, rows_b, sem_b)
      fetch_src(c0 + 2, src_ia, sem_ia)
      scatter(c0, rows_a)
      wait_gather(src_ib, rows_b, sem_b)
      wait_src(src_ia, sem_ia)
      gather(src_ia, rows_a, sem_a)
      fetch_src(jnp.minimum(c0 + 3, NCHUNK - 1), src_ib, sem_ib)
      scatter(c0 + 1, rows_b)
      return carry

    lax.fori_loop(0, (NCHUNK - 1) // 2, pair_body, 0)
    wait_gather(src_ia, rows_a, sem_a)
    scatter(NCHUNK - 1, rows_a)
    wait_src(src_ib, sem_ib)
    plsc.subcore_barrier()

    def pub_chunk(k, carry):
      r = (s + k * NS) * RCH
      pltpu.sync_copy(agg_sh.at[pl.ds(r, RCH)], rows_a)
      pltpu.sync_copy(rows_a, agg_hbm.at[pl.ds(c * N + r, RCH)])
      return carry

    lax.fori_loop(0, nch, pub_chunk, 0)

  return body(x, src, dst3, zeros_nd)


EC = 2000


def _tc_degree(dst_col):
  def body(d_ref, o_ref):
    i = pl.program_id(0)
    d = d_ref[...]
    lanes = lax.broadcasted_iota(jnp.int32, (1, 128), 1)
    hi_oh = (lax.shift_right_logical(d, 7) == lanes).astype(jnp.bfloat16)
    lo_oh = (lax.bitwise_and(d, 127) == lanes).astype(jnp.bfloat16)
    part = lax.dot_general(hi_oh, lo_oh, (((0,), (0,)), ((), ())),
                           preferred_element_type=jnp.float32)

    @pl.when(i == 0)
    def _first():
      o_ref[...] = part

    @pl.when(i > 0)
    def _rest():
      o_ref[...] += part

  return pl.pallas_call(
      body,
      grid=(E // EC,),
      in_specs=[pl.BlockSpec((EC, 1), lambda i: (i, 0))],
      out_specs=pl.BlockSpec((128, 128), lambda i: (0, 0)),
      out_shape=jax.ShapeDtypeStruct((128, 128), jnp.float32),
  )(dst_col)


BR = 1000


def _tc_combine(x, agg0, agg1, deg_col, W_self, W_neigh, b_self, b_neigh):
  def body(x_ref, a0_ref, a1_ref, d_ref, ws_ref, wn_ref,
           bs_ref, bn_ref, o_ref):
    deg = jnp.maximum(d_ref[...], 1.0)
    agg = (a0_ref[...] + a1_ref[...]) / deg
    h = lax.dot_general(x_ref[...], ws_ref[...], (((1,), (1,)), ((), ())),
                        preferred_element_type=jnp.float32)
    h = h + lax.dot_general(agg, wn_ref[...], (((1,), (1,)), ((), ())),
                            preferred_element_type=jnp.float32)
    h = h + bs_ref[...] + bn_ref[...]
    o_ref[...] = jnp.maximum(h, 0.0)

  row_blk = lambda i: (i, 0)
  fixed = lambda i: (0, 0)
  return pl.pallas_call(
      body,
      grid=(N // BR,),
      in_specs=[
          pl.BlockSpec((BR, D), row_blk),
          pl.BlockSpec((BR, D), row_blk),
          pl.BlockSpec((BR, D), row_blk),
          pl.BlockSpec((BR, 1), row_blk),
          pl.BlockSpec((D, D), fixed),
          pl.BlockSpec((D, D), fixed),
          pl.BlockSpec((1, D), fixed),
          pl.BlockSpec((1, D), fixed),
      ],
      out_specs=pl.BlockSpec((BR, D), row_blk),
      out_shape=jax.ShapeDtypeStruct((N, D), jnp.float32),
  )(x, agg0, agg1, deg_col, W_self, W_neigh,
    b_self[None, :], b_neigh[None, :])


def kernel(x, edge_index, W_self, b_self, W_neigh, b_neigh):
  src = edge_index[0]
  dst = edge_index[1]
  dst3 = dst.reshape(NW, NCHUNK, CH)
  zeros_nd = jnp.zeros((N, D), jnp.float32)
  agg_parts = _sc_aggregate(x, src, dst3, zeros_nd)
  degfold = _tc_degree(dst[:, None])
  deg_col = degfold.reshape(-1)[:N][:, None]
  return _tc_combine(x, agg_parts[:N], agg_parts[N:], deg_col,
                     W_self, W_neigh, b_self, b_neigh)


# speedup vs baseline: 6.8493x; 1.1610x over previous
import functools

import jax
import jax.numpy as jnp
from jax import lax
from jax.experimental import pallas as pl
from jax.experimental.pallas import tpu as pltpu
from jax.experimental.pallas import tpu_sc as plsc

N = 10000
E = 320000
D = 128

NC = 2
NS = 16
NW = NC * NS
EPW = E // NW
CH = 80
NCHUNK = EPW // CH
RCH = 80
NRCH = N // RCH


def _sc_aggregate(x, src, dst3, zeros_nd):
  mesh = plsc.VectorSubcoreMesh(core_axis_name="c", subcore_axis_name="s")

  @functools.partial(
      pl.kernel,
      out_type=jax.ShapeDtypeStruct((NC * N, D), jnp.float32),
      mesh=mesh,
      scratch_types=[
          pltpu.VMEM((CH,), jnp.int32),
          pltpu.VMEM((CH,), jnp.int32),
          pltpu.VMEM((NCHUNK, CH), jnp.int32),
          pltpu.VMEM((CH, D), jnp.float32),
          pltpu.VMEM((CH, D), jnp.float32),
          pltpu.VMEM_SHARED((N, D), jnp.float32),
          pltpu.SemaphoreType.DMA,
          pltpu.SemaphoreType.DMA,
          pltpu.SemaphoreType.DMA,
          pltpu.SemaphoreType.DMA,
      ],
  )
  def body(x_hbm, src_hbm, dst_hbm, znd_hbm, agg_hbm,
           src_ia, src_ib, dsts_v, rows_a, rows_b, agg_sh,
           sem_a, sem_b, sem_ia, sem_ib):
    c = lax.axis_index("c")
    s = lax.axis_index("s")
    wid = s * NC + c

    pltpu.sync_copy(dst_hbm.at[wid], dsts_v)

    pltpu.sync_copy(znd_hbm, rows_b)
    nch = 7 + jnp.where(s < NRCH - 7 * NS, 1, 0)

    def init_chunk(k, carry):
      pltpu.sync_copy(rows_b, agg_sh.at[pl.ds((s + k * NS) * RCH, RCH)])
      return carry

    lax.fori_loop(0, nch, init_chunk, 0)
    plsc.subcore_barrier()

    base0 = wid * EPW

    def fetch_src(ci, idx_v, sem):
      pltpu.async_copy(src_hbm.at[pl.ds(base0 + ci * CH, CH)], idx_v, sem)

    def wait_src(idx_v, sem):
      pltpu.make_async_copy(src_hbm.at[pl.ds(base0, CH)], idx_v, sem).wait()

    def gather(idx_v, rows, sem):
      pltpu.async_copy(x_hbm.at[idx_v], rows, sem)

    def wait_gather(idx_v, rows, sem):
      pltpu.make_async_copy(x_hbm.at[idx_v], rows, sem).wait()

    def scatter(ci, rows):
      pltpu.sync_copy(rows, agg_sh.at[dsts_v.at[ci]], add=True)

    pltpu.sync_copy(src_hbm.at[pl.ds(base0, CH)], src_ia)
    gather(src_ia, rows_a, sem_a)
    fetch_src(1, src_ib, sem_ib)

    def pair_body(j, carry):
      c0 = 2 * j
      wait_gather(src_ia, rows_a, sem_a)
      wait_src(src_ib, sem_ib)
      gather(src_ib, rows_b, sem_b)
      fetch_src(c0 + 2, src_ia, sem_ia)
      scatter(c0, rows_a)
      wait_gather(src_ib, rows_b, sem_b)
      wait_src(src_ia, sem_ia)
      gather(src_ia, rows_a, sem_a)
      fetch_src(jnp.minimum(c0 + 3, NCHUNK - 1), src_ib, sem_ib)
      scatter(c0 + 1, rows_b)
      return carry

    lax.fori_loop(0, (NCHUNK - 1) // 2, pair_body, 0)
    wait_gather(src_ia, rows_a, sem_a)
    scatter(NCHUNK - 1, rows_a)
    wait_src(src_ib, sem_ib)
    plsc.subcore_barrier()

    def pub_chunk(k, carry):
      r = (s + k * NS) * RCH
      pltpu.sync_copy(agg_sh.at[pl.ds(r, RCH)], rows_a)
      pltpu.sync_copy(rows_a, agg_hbm.at[pl.ds(c * N + r, RCH)])
      return carry

    lax.fori_loop(0, nch, pub_chunk, 0)

  return body(x, src, dst3, zeros_nd)


EC = 4000


def _tc_degree(dst_col):
  def body(d_ref, o_ref):
    i = pl.program_id(0)
    d = d_ref[...]
    lanes = lax.broadcasted_iota(jnp.int32, (1, 128), 1)
    hi_oh = (lax.shift_right_logical(d, 7) == lanes).astype(jnp.bfloat16)
    lo_oh = (lax.bitwise_and(d, 127) == lanes).astype(jnp.bfloat16)
    part = lax.dot_general(hi_oh, lo_oh, (((0,), (0,)), ((), ())),
                           preferred_element_type=jnp.float32)

    @pl.when(i == 0)
    def _first():
      o_ref[...] = part

    @pl.when(i > 0)
    def _rest():
      o_ref[...] += part

  return pl.pallas_call(
      body,
      grid=(E // EC,),
      in_specs=[pl.BlockSpec((EC, 1), lambda i: (i, 0))],
      out_specs=pl.BlockSpec((128, 128), lambda i: (0, 0)),
      out_shape=jax.ShapeDtypeStruct((128, 128), jnp.float32),
  )(dst_col)


BR = 1000


def _tc_combine(x, agg0, agg1, deg_col, W_self, W_neigh, b_self, b_neigh):
  def body(x_ref, a0_ref, a1_ref, d_ref, ws_ref, wn_ref,
           bs_ref, bn_ref, o_ref):
    deg = jnp.maximum(d_ref[...], 1.0)
    agg = (a0_ref[...] + a1_ref[...]) / deg
    h = lax.dot_general(x_ref[...], ws_ref[...], (((1,), (1,)), ((), ())),
                        preferred_element_type=jnp.float32)
    h = h + lax.dot_general(agg, wn_ref[...], (((1,), (1,)), ((), ())),
                            preferred_element_type=jnp.float32)
    h = h + bs_ref[...] + bn_ref[...]
    o_ref[...] = jnp.maximum(h, 0.0)

  row_blk = lambda i: (i, 0)
  fixed = lambda i: (0, 0)
  return pl.pallas_call(
      body,
      grid=(N // BR,),
      in_specs=[
          pl.BlockSpec((BR, D), row_blk),
          pl.BlockSpec((BR, D), row_blk),
          pl.BlockSpec((BR, D), row_blk),
          pl.BlockSpec((BR, 1), row_blk),
          pl.BlockSpec((D, D), fixed),
          pl.BlockSpec((D, D), fixed),
          pl.BlockSpec((1, D), fixed),
          pl.BlockSpec((1, D), fixed),
      ],
      out_specs=pl.BlockSpec((BR, D), row_blk),
      out_shape=jax.ShapeDtypeStruct((N, D), jnp.float32),
  )(x, agg0, agg1, deg_col, W_self, W_neigh,
    b_self[None, :], b_neigh[None, :])


def kernel(x, edge_index, W_self, b_self, W_neigh, b_neigh):
  src = edge_index[0]
  dst = edge_index[1]
  dst3 = dst.reshape(NW, NCHUNK, CH)
  zeros_nd = jnp.zeros((RCH, D), jnp.float32)
  degfold = _tc_degree(dst[:, None])
  deg_col = degfold.reshape(-1)[:N][:, None]
  agg_parts = _sc_aggregate(x, src, dst3, zeros_nd)
  return _tc_combine(x, agg_parts[:N], agg_parts[N:], deg_col,
                     W_self, W_neigh, b_self, b_neigh)


# speedup vs baseline: 7.3300x; 1.0702x over previous
import functools

import jax
import jax.numpy as jnp
from jax import lax
from jax.experimental import pallas as pl
from jax.experimental.pallas import tpu as pltpu
from jax.experimental.pallas import tpu_sc as plsc

N = 10000
E = 320000
D = 128

NC = 2
NS = 16
NW = NC * NS
EPW = E // NW
CH = 80
NCHUNK = EPW // CH
RCH = 80
NRCH = N // RCH


def _sc_aggregate(x, src, dst3, zeros_nd):
  mesh = plsc.VectorSubcoreMesh(core_axis_name="c", subcore_axis_name="s")

  @functools.partial(
      pl.kernel,
      out_type=jax.ShapeDtypeStruct((NC * N, D), jnp.float32),
      mesh=mesh,
      scratch_types=[
          pltpu.VMEM((CH,), jnp.int32),
          pltpu.VMEM((CH,), jnp.int32),
          pltpu.VMEM((NCHUNK, CH), jnp.int32),
          pltpu.VMEM((CH, D), jnp.float32),
          pltpu.VMEM((CH, D), jnp.float32),
          pltpu.VMEM_SHARED((N, D), jnp.float32),
          pltpu.SemaphoreType.DMA,
          pltpu.SemaphoreType.DMA,
          pltpu.SemaphoreType.DMA,
          pltpu.SemaphoreType.DMA,
      ],
  )
  def body(x_hbm, src_hbm, dst_hbm, znd_hbm, agg_hbm,
           src_ia, src_ib, dsts_v, rows_a, rows_b, agg_sh,
           sem_a, sem_b, sem_ia, sem_ib):
    c = lax.axis_index("c")
    s = lax.axis_index("s")
    wid = s * NC + c

    pltpu.sync_copy(dst_hbm.at[wid], dsts_v)

    pltpu.sync_copy(znd_hbm, rows_b)
    nch = 7 + jnp.where(s < NRCH - 7 * NS, 1, 0)

    def init_chunk(k, carry):
      pltpu.sync_copy(rows_b, agg_sh.at[pl.ds((s + k * NS) * RCH, RCH)])
      return carry

    lax.fori_loop(0, nch, init_chunk, 0)
    plsc.subcore_barrier()

    base0 = wid * EPW

    def fetch_src(ci, idx_v, sem):
      pltpu.async_copy(src_hbm.at[pl.ds(base0 + ci * CH, CH)], idx_v, sem)

    def wait_src(idx_v, sem):
      pltpu.make_async_copy(src_hbm.at[pl.ds(base0, CH)], idx_v, sem).wait()

    def gather(idx_v, rows, sem):
      pltpu.async_copy(x_hbm.at[idx_v], rows, sem)

    def wait_gather(idx_v, rows, sem):
      pltpu.make_async_copy(x_hbm.at[idx_v], rows, sem).wait()

    def scatter(ci, rows):
      pltpu.sync_copy(rows, agg_sh.at[dsts_v.at[ci]], add=True)

    pltpu.sync_copy(src_hbm.at[pl.ds(base0, CH)], src_ia)
    gather(src_ia, rows_a, sem_a)
    fetch_src(1, src_ib, sem_ib)

    def pair_body(j, carry):
      c0 = 2 * j
      wait_gather(src_ia, rows_a, sem_a)
      wait_src(src_ib, sem_ib)
      gather(src_ib, rows_b, sem_b)
      fetch_src(c0 + 2, src_ia, sem_ia)
      scatter(c0, rows_a)
      wait_gather(src_ib, rows_b, sem_b)
      wait_src(src_ia, sem_ia)
      gather(src_ia, rows_a, sem_a)
      fetch_src(jnp.minimum(c0 + 3, NCHUNK - 1), src_ib, sem_ib)
      scatter(c0 + 1, rows_b)
      return carry

    lax.fori_loop(0, (NCHUNK - 1) // 2, pair_body, 0)
    wait_gather(src_ia, rows_a, sem_a)
    scatter(NCHUNK - 1, rows_a)
    wait_src(src_ib, sem_ib)
    plsc.subcore_barrier()

    def pub_chunk(k, carry):
      r = (s + k * NS) * RCH
      pltpu.sync_copy(agg_sh.at[pl.ds(r, RCH)], rows_a)
      pltpu.sync_copy(rows_a, agg_hbm.at[pl.ds(c * N + r, RCH)])
      return carry

    lax.fori_loop(0, nch, pub_chunk, 0)

  return body(x, src, dst3, zeros_nd)


EC = 8000


def _tc_degree(dst_col):
  def body(d_ref, o_ref):
    i = pl.program_id(0)
    d = d_ref[...]
    lanes = lax.broadcasted_iota(jnp.int32, (1, 128), 1)
    hi_oh = (lax.shift_right_logical(d, 7) == lanes).astype(jnp.bfloat16)
    lo_oh = (lax.bitwise_and(d, 127) == lanes).astype(jnp.bfloat16)
    part = lax.dot_general(hi_oh, lo_oh, (((0,), (0,)), ((), ())),
                           preferred_element_type=jnp.float32)

    @pl.when(i == 0)
    def _first():
      o_ref[...] = part

    @pl.when(i > 0)
    def _rest():
      o_ref[...] += part

  return pl.pallas_call(
      body,
      grid=(E // EC,),
      in_specs=[pl.BlockSpec((EC, 1), lambda i: (i, 0))],
      out_specs=pl.BlockSpec((128, 128), lambda i: (0, 0)),
      out_shape=jax.ShapeDtypeStruct((128, 128), jnp.float32),
  )(dst_col)


BR = 1000


def _tc_combine(x, agg0, agg1, deg_col, W_self, W_neigh, b_self, b_neigh):
  def body(x_ref, a0_ref, a1_ref, d_ref, ws_ref, wn_ref,
           bs_ref, bn_ref, o_ref):
    deg = jnp.maximum(d_ref[...], 1.0)
    agg = (a0_ref[...] + a1_ref[...]) / deg
    h = lax.dot_general(x_ref[...], ws_ref[...], (((1,), (1,)), ((), ())),
                        preferred_element_type=jnp.float32)
    h = h + lax.dot_general(agg, wn_ref[...], (((1,), (1,)), ((), ())),
                            preferred_element_type=jnp.float32)
    h = h + bs_ref[...] + bn_ref[...]
    o_ref[...] = jnp.maximum(h, 0.0)

  row_blk = lambda i: (i, 0)
  fixed = lambda i: (0, 0)
  return pl.pallas_call(
      body,
      grid=(N // BR,),
      in_specs=[
          pl.BlockSpec((BR, D), row_blk),
          pl.BlockSpec((BR, D), row_blk),
          pl.BlockSpec((BR, D), row_blk),
          pl.BlockSpec((BR, 1), row_blk),
          pl.BlockSpec((D, D), fixed),
          pl.BlockSpec((D, D), fixed),
          pl.BlockSpec((1, D), fixed),
          pl.BlockSpec((1, D), fixed),
      ],
      out_specs=pl.BlockSpec((BR, D), row_blk),
      out_shape=jax.ShapeDtypeStruct((N, D), jnp.float32),
  )(x, agg0, agg1, deg_col, W_self, W_neigh,
    b_self[None, :], b_neigh[None, :])


def kernel(x, edge_index, W_self, b_self, W_neigh, b_neigh):
  src = edge_index[0]
  dst = edge_index[1]
  dst3 = dst.reshape(NW, NCHUNK, CH)
  zeros_nd = jnp.zeros((RCH, D), jnp.float32)
  degfold = _tc_degree(dst[:, None])
  deg_col = degfold.reshape(-1)[:N][:, None]
  agg_parts = _sc_aggregate(x, src, dst3, zeros_nd)
  return _tc_combine(x, agg_parts[:N], agg_parts[N:], deg_col,
                     W_self, W_neigh, b_self, b_neigh)


# speedup vs baseline: 7.3664x; 1.0050x over previous
import functools

import jax
import jax.numpy as jnp
from jax import lax
from jax.experimental import pallas as pl
from jax.experimental.pallas import tpu as pltpu
from jax.experimental.pallas import tpu_sc as plsc

N = 10000
E = 320000
D = 128

NC = 2
NS = 16
NW = NC * NS
EPW = E // NW
CH = 80
NCHUNK = EPW // CH
RCH = 80
NRCH = N // RCH


def _sc_aggregate(x, src, dst3, zeros_nd):
  mesh = plsc.VectorSubcoreMesh(core_axis_name="c", subcore_axis_name="s")

  @functools.partial(
      pl.kernel,
      out_type=jax.ShapeDtypeStruct((NC * N, D), jnp.float32),
      mesh=mesh,
      scratch_types=[
          pltpu.VMEM((CH,), jnp.int32),
          pltpu.VMEM((CH,), jnp.int32),
          pltpu.VMEM((NCHUNK, CH), jnp.int32),
          pltpu.VMEM((CH, D), jnp.float32),
          pltpu.VMEM((CH, D), jnp.float32),
          pltpu.VMEM_SHARED((N, D), jnp.float32),
          pltpu.SemaphoreType.DMA,
          pltpu.SemaphoreType.DMA,
          pltpu.SemaphoreType.DMA,
          pltpu.SemaphoreType.DMA,
      ],
  )
  def body(x_hbm, src_hbm, dst_hbm, znd_hbm, agg_hbm,
           src_ia, src_ib, dsts_v, rows_a, rows_b, agg_sh,
           sem_a, sem_b, sem_ia, sem_ib):
    c = lax.axis_index("c")
    s = lax.axis_index("s")
    wid = s * NC + c

    pltpu.sync_copy(dst_hbm.at[wid], dsts_v)

    pltpu.sync_copy(znd_hbm, rows_b)
    nch = 7 + jnp.where(s < NRCH - 7 * NS, 1, 0)

    def init_chunk(k, carry):
      pltpu.sync_copy(rows_b, agg_sh.at[pl.ds((s + k * NS) * RCH, RCH)])
      return carry

    lax.fori_loop(0, nch, init_chunk, 0)
    plsc.subcore_barrier()

    base0 = wid * EPW

    def fetch_src(ci, idx_v, sem):
      pltpu.async_copy(src_hbm.at[pl.ds(base0 + ci * CH, CH)], idx_v, sem)

    def wait_src(idx_v, sem):
      pltpu.make_async_copy(src_hbm.at[pl.ds(base0, CH)], idx_v, sem).wait()

    def gather(idx_v, rows, sem):
      pltpu.async_copy(x_hbm.at[idx_v], rows, sem)

    def wait_gather(idx_v, rows, sem):
      pltpu.make_async_copy(x_hbm.at[idx_v], rows, sem).wait()

    def scatter(ci, rows):
      pltpu.sync_copy(rows, agg_sh.at[dsts_v.at[ci]], add=True)

    pltpu.sync_copy(src_hbm.at[pl.ds(base0, CH)], src_ia)
    gather(src_ia, rows_a, sem_a)
    fetch_src(1, src_ib, sem_ib)

    def pair_body(j, carry):
      c0 = 2 * j
      wait_gather(src_ia, rows_a, sem_a)
      wait_src(src_ib, sem_ib)
      gather(src_ib, rows_b, sem_b)
      fetch_src(c0 + 2, src_ia, sem_ia)
      scatter(c0, rows_a)
      wait_gather(src_ib, rows_b, sem_b)
      wait_src(src_ia, sem_ia)
      gather(src_ia, rows_a, sem_a)
      fetch_src(jnp.minimum(c0 + 3, NCHUNK - 1), src_ib, sem_ib)
      scatter(c0 + 1, rows_b)
      return carry

    lax.fori_loop(0, (NCHUNK - 1) // 2, pair_body, 0)
    wait_gather(src_ia, rows_a, sem_a)
    scatter(NCHUNK - 1, rows_a)
    wait_src(src_ib, sem_ib)
    plsc.subcore_barrier()

    def pub_chunk(k, carry):
      r = (s + k * NS) * RCH
      pltpu.sync_copy(agg_sh.at[pl.ds(r, RCH)], rows_a)
      pltpu.sync_copy(rows_a, agg_hbm.at[pl.ds(c * N + r, RCH)])
      return carry

    lax.fori_loop(0, nch, pub_chunk, 0)

  return body(x, src, dst3, zeros_nd)


EC = 16000


def _tc_degree(dst_col):
  def body(d_ref, o_ref):
    i = pl.program_id(0)
    d = d_ref[...]
    lanes = lax.broadcasted_iota(jnp.int32, (1, 128), 1)
    hi_oh = (lax.shift_right_logical(d, 7) == lanes).astype(jnp.bfloat16)
    lo_oh = (lax.bitwise_and(d, 127) == lanes).astype(jnp.bfloat16)
    part = lax.dot_general(hi_oh, lo_oh, (((0,), (0,)), ((), ())),
                           preferred_element_type=jnp.float32)

    @pl.when(i == 0)
    def _first():
      o_ref[...] = part

    @pl.when(i > 0)
    def _rest():
      o_ref[...] += part

  return pl.pallas_call(
      body,
      grid=(E // EC,),
      in_specs=[pl.BlockSpec((EC, 1), lambda i: (i, 0))],
      out_specs=pl.BlockSpec((128, 128), lambda i: (0, 0)),
      out_shape=jax.ShapeDtypeStruct((128, 128), jnp.float32),
  )(dst_col)


BR = 1000


def _tc_combine(x, agg0, agg1, deg_col, W_self, W_neigh, b_self, b_neigh):
  def body(x_ref, a0_ref, a1_ref, d_ref, ws_ref, wn_ref,
           bs_ref, bn_ref, o_ref):
    deg = jnp.maximum(d_ref[...], 1.0)
    agg = (a0_ref[...] + a1_ref[...]) / deg
    h = lax.dot_general(x_ref[...], ws_ref[...], (((1,), (1,)), ((), ())),
                        preferred_element_type=jnp.float32)
    h = h + lax.dot_general(agg, wn_ref[...], (((1,), (1,)), ((), ())),
                            preferred_element_type=jnp.float32)
    h = h + bs_ref[...] + bn_ref[...]
    o_ref[...] = jnp.maximum(h, 0.0)

  row_blk = lambda i: (i, 0)
  fixed = lambda i: (0, 0)
  return pl.pallas_call(
      body,
      grid=(N // BR,),
      in_specs=[
          pl.BlockSpec((BR, D), row_blk),
          pl.BlockSpec((BR, D), row_blk),
          pl.BlockSpec((BR, D), row_blk),
          pl.BlockSpec((BR, 1), row_blk),
          pl.BlockSpec((D, D), fixed),
          pl.BlockSpec((D, D), fixed),
          pl.BlockSpec((1, D), fixed),
          pl.BlockSpec((1, D), fixed),
      ],
      out_specs=pl.BlockSpec((BR, D), row_blk),
      out_shape=jax.ShapeDtypeStruct((N, D), jnp.float32),
  )(x, agg0, agg1, deg_col, W_self, W_neigh,
    b_self[None, :], b_neigh[None, :])


def kernel(x, edge_index, W_self, b_self, W_neigh, b_neigh):
  src = edge_index[0]
  dst = edge_index[1]
  dst3 = dst.reshape(NW, NCHUNK, CH)
  zeros_nd = jnp.zeros((RCH, D), jnp.float32)
  degfold = _tc_degree(dst[:, None])
  deg_col = degfold.reshape(-1)[:N][:, None]
  agg_parts = _sc_aggregate(x, src, dst3, zeros_nd)
  return _tc_combine(x, agg_parts[:N], agg_parts[N:], deg_col,
                     W_self, W_neigh, b_self, b_neigh)
